# Initial kernel scaffold; baseline (speedup 1.0000x reference)
import jax
import jax.numpy as jnp
from jax.experimental import pallas as pl


def kernel(tensors, mask, Wg, We, be, Wp, bp):
    raise NotImplementedError("write your pallas kernel here")


# speedup vs baseline: 4.5313x; 4.5313x over previous
import jax
import jax.numpy as jnp
from jax.experimental import pallas as pl
from jax.experimental.pallas import tpu as pltpu

B, L, N = 4, 3, 2048
D = 1024
H = 4096
E = 3
S = B * L * N
TPB = L * N
CAP = S // E
TBLK = 2048
NBLK = S // TBLK
BPB = TPB // TBLK


def _cumsum_rows(a):
    n = a.shape[0]
    k = 1
    while k < n:
        shifted = jnp.concatenate(
            [jnp.zeros((k, a.shape[1]), a.dtype), a[:-k, :]], axis=0)
        a = a + shifted
        k *= 2
    return a


def _pass1_body(x_ref, wg_ref, s_ref, st_ref):
    i = pl.program_id(0)

    @pl.when(i == 0)
    def _init():
        s_ref[...] = jnp.zeros_like(s_ref)
        st_ref[...] = jnp.zeros_like(st_ref)

    x = x_ref[...]
    wg = wg_ref[...]
    logits = jnp.dot(x, wg, preferred_element_type=jnp.float32)
    mx = jnp.max(logits, axis=1, keepdims=True)
    ex = jnp.exp(logits - mx)
    den = jnp.sum(ex, axis=1, keepdims=True)
    probs = ex / den
    pmax = jnp.max(probs, axis=1, keepdims=True)
    cols = jax.lax.broadcasted_iota(jnp.int32, (TBLK, E), 1)
    eidx = jnp.min(jnp.where(probs >= pmax, cols, E), axis=1, keepdims=True)
    onehot = (cols == eidx).astype(jnp.float32)

    carry = st_ref[1:2, 0:E]
    csum = _cumsum_rows(onehot)
    pos = jnp.sum(onehot * (csum - 1.0 + carry), axis=1, keepdims=True)
    keep = (pos < float(CAP)).astype(jnp.float32)
    gk = pmax * keep

    b = i // BPB
    cols16 = jax.lax.broadcasted_iota(jnp.int32, (TBLK, 16), 1)
    w_full = jnp.where(cols16 == (eidx * B + b), gk, 0.0)
    s_ref[...] += jax.lax.dot_general(
        w_full, x, (((0,), (0,)), ((), ())),
        preferred_element_type=jnp.float32)

    st_ref[0:1, 0:E] += jnp.sum(probs, axis=0, keepdims=True)
    st_ref[1:2, 0:E] += csum[TBLK - 1:TBLK, :]
    st_ref[pl.ds(4 + b, 1), 0:E] += jnp.sum(onehot * gk, axis=0, keepdims=True)


def _finalize_body(s_ref, st_ref, we_ref, be_ref, wp_ref, bp_ref, mask_ref,
                   pooled_ref, aux_ref):
    s = s_ref[...]
    acc = jnp.zeros((B, D), jnp.float32)
    for e in range(E):
        acc += jnp.dot(s[e * B:(e + 1) * B, :], we_ref[e, :, :],
                       preferred_element_type=jnp.float32)
    acc += jnp.dot(st_ref[4:8, 0:E], be_ref[...],
                   preferred_element_type=jnp.float32)
    py = jnp.dot(acc, wp_ref[...], preferred_element_type=jnp.float32)
    py = py + float(TPB) * bp_ref[...]
    valid = float(TPB) - jnp.sum(mask_ref[...], axis=1, keepdims=True)
    cnt = jnp.maximum(valid, 1.0)
    pooled_ref[...] = py / cnt
    probsum = st_ref[0:1, 0:E]
    cnts = st_ref[1:2, 0:E]
    aux = (float(E) / (float(S) * float(S))) * jnp.sum(probsum * cnts)
    aux_ref[...] = jnp.full((1, 1), aux, jnp.float32)


def kernel(tensors, mask, Wg, We, be, Wp, bp):
    x = jnp.transpose(tensors, (0, 2, 1, 3)).reshape(S, D)
    maskf = mask.reshape(B, TPB).astype(jnp.float32)
    s, st = pl.pallas_call(
        _pass1_body,
        grid=(NBLK,),
        in_specs=[pl.BlockSpec((TBLK, D), lambda i: (i, 0)),
                  pl.BlockSpec((D, E), lambda i: (0, 0))],
        out_specs=[pl.BlockSpec((16, D), lambda i: (0, 0)),
                   pl.BlockSpec((8, 128), lambda i: (0, 0))],
        out_shape=[jax.ShapeDtypeStruct((16, D), jnp.float32),
                   jax.ShapeDtypeStruct((8, 128), jnp.float32)],
        compiler_params=pltpu.CompilerParams(
            dimension_semantics=("arbitrary",)),
    )(x, Wg)
    pooled, aux = pl.pallas_call(
        _finalize_body,
        in_specs=[pl.BlockSpec((16, D), lambda: (0, 0)),
                  pl.BlockSpec((8, 128), lambda: (0, 0)),
                  pl.BlockSpec((E, D, D), lambda: (0, 0, 0)),
                  pl.BlockSpec((E, D), lambda: (0, 0)),
                  pl.BlockSpec((D, H), lambda: (0, 0)),
                  pl.BlockSpec((1, H), lambda: (0, 0)),
                  pl.BlockSpec((B, TPB), lambda: (0, 0))],
        out_specs=[pl.BlockSpec((B, H), lambda: (0, 0)),
                   pl.BlockSpec((1, 1), lambda: (0, 0))],
        out_shape=[jax.ShapeDtypeStruct((B, H), jnp.float32),
                   jax.ShapeDtypeStruct((1, 1), jnp.float32)],
    )(s, st, We, be, Wp, bp.reshape(1, H), maskf)
    return pooled, aux[0, 0]


# speedup vs baseline: 18.8125x; 4.1517x over previous
import jax
import jax.numpy as jnp
from jax.experimental import pallas as pl
from jax.experimental.pallas import tpu as pltpu

B, L, N = 4, 3, 2048
D = 1024
H = 4096
E = 3
S = B * L * N
TPB = L * N
CAP = S // E
TN = 512
NB = N // TN
TBLK = L * TN


def _cumsum_rows(a):
    n = a.shape[0]
    k = 1
    while k < n:
        shifted = jnp.concatenate(
            [jnp.zeros((k, a.shape[1]), a.dtype), a[:-k, :]], axis=0)
        a = a + shifted
        k *= 2
    return a


def _pass1_body(x_ref, wg_ref, s_ref, st_ref):
    b = pl.program_id(0)
    j = pl.program_id(1)

    @pl.when(jnp.logical_and(b == 0, j == 0))
    def _init():
        s_ref[...] = jnp.zeros_like(s_ref)
        st_ref[...] = jnp.zeros_like(st_ref)

    x = x_ref[...]
    x2 = x.reshape(TBLK, D)
    wg = wg_ref[...]
    logits = jnp.dot(x2, wg, preferred_element_type=jnp.float32)
    mx = jnp.max(logits, axis=1, keepdims=True)
    ex = jnp.exp(logits - mx)
    den = jnp.sum(ex, axis=1, keepdims=True)
    probs = ex / den
    pmax = jnp.max(probs, axis=1, keepdims=True)
    cols = jax.lax.broadcasted_iota(jnp.int32, (TBLK, E), 1)
    eidx = jnp.min(jnp.where(probs >= pmax, cols, E), axis=1, keepdims=True)
    onehot = (cols == eidx).astype(jnp.float32)

    carry = st_ref[1:2, 0:E]
    oh3 = onehot.reshape(L, TN, E)
    col_tot = jnp.sum(oh3, axis=0)
    excl_n = _cumsum_rows(col_tot) - col_tot
    cs_l = oh3
    cs_l = cs_l + jnp.concatenate(
        [jnp.zeros((1, TN, E), jnp.float32), cs_l[:-1]], axis=0)
    cs_l = cs_l + jnp.concatenate(
        [jnp.zeros((2, TN, E), jnp.float32), cs_l[:-2]], axis=0)
    excl_l = cs_l - oh3
    pos3 = excl_n[None, :, :] + excl_l + carry.reshape(1, 1, E)
    pos = jnp.sum(oh3 * pos3, axis=2)
    keep = (pos < float(CAP)).astype(jnp.float32).reshape(TBLK, 1)
    gk = pmax * keep

    cols16 = jax.lax.broadcasted_iota(jnp.int32, (TBLK, 16), 1)
    w_full = jnp.where(cols16 == (eidx * B + b), gk, 0.0)
    s_ref[...] += jax.lax.dot_general(
        w_full, x2, (((0,), (0,)), ((), ())),
        preferred_element_type=jnp.float32)

    st_ref[0:1, 0:E] += jnp.sum(probs, axis=0, keepdims=True)
    st_ref[1:2, 0:E] += jnp.sum(onehot, axis=0, keepdims=True)
    st_ref[pl.ds(4 + b, 1), 0:E] += jnp.sum(onehot * gk, axis=0, keepdims=True)


def _finalize_body(s_ref, st_ref, we_ref, be_ref, wp_ref, bp_ref, mask_ref,
                   pooled_ref, aux_ref):
    s = s_ref[...]
    acc = jnp.zeros((B, D), jnp.float32)
    for e in range(E):
        acc += jnp.dot(s[e * B:(e + 1) * B, :], we_ref[e, :, :],
                       preferred_element_type=jnp.float32)
    acc += jnp.dot(st_ref[4:8, 0:E], be_ref[...],
                   preferred_element_type=jnp.float32)
    py = jnp.dot(acc, wp_ref[...], preferred_element_type=jnp.float32)
    py = py + float(TPB) * bp_ref[...]
    valid = float(TPB) - jnp.sum(mask_ref[...], axis=1, keepdims=True)
    cnt = jnp.maximum(valid, 1.0)
    pooled_ref[...] = py / cnt
    probsum = st_ref[0:1, 0:E]
    cnts = st_ref[1:2, 0:E]
    aux = (float(E) / (float(S) * float(S))) * jnp.sum(probsum * cnts)
    aux_ref[...] = jnp.full((1, 1), aux, jnp.float32)


def kernel(tensors, mask, Wg, We, be, Wp, bp):
    maskf = mask.reshape(B, TPB).astype(jnp.float32)
    s, st = pl.pallas_call(
        _pass1_body,
        grid=(B, NB),
        in_specs=[pl.BlockSpec((1, L, TN, D), lambda b, j: (b, 0, j, 0)),
                  pl.BlockSpec((D, E), lambda b, j: (0, 0))],
        out_specs=[pl.BlockSpec((16, D), lambda b, j: (0, 0)),
                   pl.BlockSpec((8, 128), lambda b, j: (0, 0))],
        out_shape=[jax.ShapeDtypeStruct((16, D), jnp.float32),
                   jax.ShapeDtypeStruct((8, 128), jnp.float32)],
        compiler_params=pltpu.CompilerParams(
            dimension_semantics=("arbitrary", "arbitrary")),
    )(tensors, Wg)
    pooled, aux = pl.pallas_call(
        _finalize_body,
        in_specs=[pl.BlockSpec((16, D), lambda: (0, 0)),
                  pl.BlockSpec((8, 128), lambda: (0, 0)),
                  pl.BlockSpec((E, D, D), lambda: (0, 0, 0)),
                  pl.BlockSpec((E, D), lambda: (0, 0)),
                  pl.BlockSpec((D, H), lambda: (0, 0)),
                  pl.BlockSpec((1, H), lambda: (0, 0)),
                  pl.BlockSpec((B, TPB), lambda: (0, 0))],
        out_specs=[pl.BlockSpec((B, H), lambda: (0, 0)),
                   pl.BlockSpec((1, 1), lambda: (0, 0))],
        out_shape=[jax.ShapeDtypeStruct((B, H), jnp.float32),
                   jax.ShapeDtypeStruct((1, 1), jnp.float32)],
    )(s, st, We, be, Wp, bp.reshape(1, H), maskf)
    return pooled, aux[0, 0]
